# baseline (speedup 1.0000x reference)
import jax
import jax.numpy as jnp
from jax.experimental import pallas as pl
from jax.experimental.pallas import tpu as pltpu
from jax.experimental.pallas import tpu_sc as plsc

_B, _N, _S, _C, _NS = 2, 10000, 512, 512, 16
_RADIUS, _HMIN, _HMAX = 0.05, -0.02, 0.04
_NO = 480
_LANES = 128
_NCH = (_N + _LANES - 1) // _LANES
_NP = _NCH * _LANES
_SB = 128
_MB = 64
_BIG = 2 ** 30
_GW = 64


def _bf(a):
    return a.astype(jnp.bfloat16).astype(jnp.float32)


def _query_body(xyz_ref, nc_ref, vr_ref, out_ref, keys_ref):
    b = pl.program_id(0)
    cx = nc_ref[0, :, 0:1]
    cy = nc_ref[0, :, 1:2]
    cz = nc_ref[0, :, 2:3]
    vr = [_bf(vr_ref[0, :, k:k + 1]) for k in range(9)]
    r2 = jnp.float32(_RADIUS * _RADIUS)
    for c in range(_NCH):
        px = xyz_ref[0, 0, c, :][None, :]
        py = xyz_ref[0, 1, c, :][None, :]
        pz = xyz_ref[0, 2, c, :][None, :]
        relx = _bf(px - cx)
        rely = _bf(py - cy)
        relz = _bf(pz - cz)
        rx = vr[0] * relx + vr[1] * rely + vr[2] * relz
        ry = vr[3] * relx + vr[4] * rely + vr[5] * relz
        rz = vr[6] * relx + vr[7] * rely + vr[8] * relz
        d2 = ry * ry + rz * rz
        iota = jax.lax.broadcasted_iota(jnp.int32, (_SB, _LANES), 1) + c * _LANES
        mask = (d2 <= r2) & (rx >= _HMIN) & (rx <= _HMAX) & (iota < _N)
        keys_ref[:, c * _LANES:(c + 1) * _LANES] = jnp.where(mask, iota, _BIG)
    m = jnp.full((_SB, 1), -1, jnp.int32)
    cols = []
    for _ in range(_NS):
        kall = keys_ref[:, :]
        kk = jnp.where(kall > m, kall, _BIG)
        m = jnp.min(kk, axis=1, keepdims=True)
        cols.append(m)
    K = jnp.concatenate(cols, axis=1)
    fb = jnp.where(K[:, 0:1] < _BIG, K[:, 0:1], 0)
    idx = jnp.where(K < _BIG, K, fb) + b * _N
    out_ref[0] = idx


def _cylinder_query(xyzr, ncz, vr9):
    return pl.pallas_call(
        _query_body,
        grid=(_B, _S // _SB),
        in_specs=[
            pl.BlockSpec((1, 3, _NCH, _LANES), lambda b, j: (b, 0, 0, 0)),
            pl.BlockSpec((1, _SB, 3), lambda b, j: (b, j, 0)),
            pl.BlockSpec((1, _SB, 9), lambda b, j: (b, j, 0)),
        ],
        out_specs=pl.BlockSpec((1, _SB, _NS), lambda b, j: (b, j, 0)),
        out_shape=jax.ShapeDtypeStruct((_B, _S, _NS), jnp.int32),
        scratch_shapes=[pltpu.VMEM((_SB, _NP), jnp.int32)],
    )(xyzr, ncz, vr9)


def _sc_gather(idx_flat, feat_flat, xyz_pad):
    nidx = idx_flat.shape[0]
    nw = 32
    bpw = nidx // nw
    nch = bpw // _GW
    mesh = plsc.VectorSubcoreMesh(core_axis_name="c", subcore_axis_name="s")

    @pl.kernel(
        out_type=(
            jax.ShapeDtypeStruct((nidx, _C // 2), jnp.int32),
            jax.ShapeDtypeStruct((nidx, 128), jnp.float32),
        ),
        mesh=mesh,
        scratch_types=[
            pltpu.VMEM((bpw,), jnp.int32),
            pltpu.VMEM((_GW, _C // 2), jnp.int32),
            pltpu.VMEM((_GW, 128), jnp.float32),
            pltpu.SemaphoreType.DMA,
            pltpu.SemaphoreType.DMA,
        ],
    )
    def k(feat_hbm, xyz_hbm, i_hbm, of_hbm, ox_hbm,
          idx_v, feat_v, xyz_v, sem_f, sem_x):
        wid = jax.lax.axis_index("s") * 2 + jax.lax.axis_index("c")
        base = wid * bpw
        pltpu.sync_copy(i_hbm.at[pl.ds(base, bpw)], idx_v)
        for c in range(nch):
            isub = idx_v.at[pl.ds(c * _GW, _GW)]
            cf = pltpu.async_copy(feat_hbm.at[isub], feat_v, sem_f)
            cx = pltpu.async_copy(xyz_hbm.at[isub], xyz_v, sem_x)
            cf.wait()
            cx.wait()
            pltpu.sync_copy(feat_v, of_hbm.at[pl.ds(base + c * _GW, _GW)])
            pltpu.sync_copy(xyz_v, ox_hbm.at[pl.ds(base + c * _GW, _GW)])

    return k(feat_flat, xyz_pad, idx_flat)


def _mlp_body(gf_ref, gx_ref, nc_ref, vr_ref,
              w0x_ref, w0f_ref, b0_ref, w1_ref, b1_ref, w2_ref, b2_ref,
              c1_ref, d1_ref, c2_ref, d2_ref, c3_ref, d3_ref,
              preds_ref, v_ref, bg_ref):
    hp = jax.lax.Precision.DEFAULT
    inv_r = jnp.float32(1.0 / _RADIUS)
    rel = [_bf(gx_ref[:, i:i + 1] - nc_ref[:, i:i + 1]) for i in range(3)]
    gxn = []
    for i in range(3):
        rot = (_bf(vr_ref[:, 3 * i + 0:3 * i + 1]) * rel[0]
               + _bf(vr_ref[:, 3 * i + 1:3 * i + 2]) * rel[1]
               + _bf(vr_ref[:, 3 * i + 2:3 * i + 3]) * rel[2])
        gxn.append(_bf(rot * inv_r))
    h = jnp.dot(gf_ref[...], w0f_ref[...], precision=hp,
                preferred_element_type=jnp.float32)
    h = h + b0_ref[...]
    for i in range(3):
        h = h + gxn[i] * _bf(w0x_ref[i, :][None, :])
    h = jax.nn.relu(h)
    h = jax.nn.relu(jnp.dot(h, w1_ref[...], precision=hp,
                            preferred_element_type=jnp.float32) + b1_ref[...])
    h = jax.nn.relu(jnp.dot(h, w2_ref[...], precision=hp,
                            preferred_element_type=jnp.float32) + b2_ref[...])
    bg = jnp.max(h.reshape(_MB, _NS, _C), axis=1)
    bg_ref[...] = bg
    v1 = jax.nn.relu(jnp.dot(bg, c1_ref[...], precision=hp,
                             preferred_element_type=jnp.float32) + d1_ref[...])
    v2 = jax.nn.relu(jnp.dot(v1, c2_ref[...], precision=hp,
                             preferred_element_type=jnp.float32) + d2_ref[...])
    v_ref[...] = v2
    preds_ref[...] = jnp.dot(v2, c3_ref[...], precision=hp,
                             preferred_element_type=jnp.float32) + d3_ref[...]


def _mlp(gfeat, gxyz, ncf, vrf, ws):
    bs = _B * _S
    nb = bs // _MB
    full = lambda a: pl.BlockSpec(a.shape, lambda i: (0,) * a.ndim)
    (w0x, w0f, b0, w1, b1, w2, b2, c1, d1, c2, d2, c3, d3) = ws
    return pl.pallas_call(
        _mlp_body,
        grid=(nb,),
        in_specs=[
            pl.BlockSpec((_MB * _NS, _C), lambda i: (i, 0)),
            pl.BlockSpec((_MB * _NS, 128), lambda i: (i, 0)),
            pl.BlockSpec((_MB * _NS, 3), lambda i: (i, 0)),
            pl.BlockSpec((_MB * _NS, 9), lambda i: (i, 0)),
            full(w0x), full(w0f), full(b0), full(w1), full(b1),
            full(w2), full(b2), full(c1), full(d1), full(c2), full(d2),
            full(c3), full(d3),
        ],
        out_specs=[
            pl.BlockSpec((_MB, _NO), lambda i: (i, 0)),
            pl.BlockSpec((_MB, _C), lambda i: (i, 0)),
            pl.BlockSpec((_MB, _C), lambda i: (i, 0)),
        ],
        out_shape=[
            jax.ShapeDtypeStruct((bs, _NO), jnp.float32),
            jax.ShapeDtypeStruct((bs, _C), jnp.float32),
            jax.ShapeDtypeStruct((bs, _C), jnp.float32),
        ],
    )(gfeat, gxyz, ncf, vrf, w0x, w0f, b0, w1, b1, w2, b2,
      c1, d1, c2, d2, c3, d3)


def _fold_weights(params):
    def fold(w, b, g, beta):
        return w * g[:, None], (b * g + beta)[None, :]

    w0, b0 = fold(params['mlp_W0'], params['mlp_b0'],
                  params['mlp_g0'], params['mlp_beta0'])
    w1, b1 = fold(params['mlp_W1'], params['mlp_b1'],
                  params['mlp_g1'], params['mlp_beta1'])
    w2, b2 = fold(params['mlp_W2'], params['mlp_b2'],
                  params['mlp_g2'], params['mlp_beta2'])
    c1, d1 = fold(params['c1_W'], params['c1_b'],
                  params['bn1_g'], params['bn1_b'])
    c2, d2 = fold(params['c2_W'], params['c2_b'],
                  params['bn2_g'], params['bn2_b'])
    return (w0[:, :3].T, w0[:, 3:].T.astype(jnp.bfloat16), b0, w1.T, b1, w2.T, b2,
            c1.T, d1, c2.T, d2,
            params['c3_W'].T, params['c3_b'][None, :])


def kernel(xyz, new_xyz, view_rot, features, params):
    xyzr = jnp.pad(xyz, ((0, 0), (0, _NP - _N), (0, 0)),
                   constant_values=1e6).swapaxes(1, 2).reshape(_B, 3, _NCH, _LANES)
    ncz = new_xyz
    vr9 = view_rot.reshape(_B, _S, 9)
    feat_bf = features.swapaxes(1, 2).astype(jnp.bfloat16)
    feat_flat = jax.lax.bitcast_convert_type(
        feat_bf.reshape(_B * _N, _C // 2, 2), jnp.int32)
    xyz_pad = jnp.pad(xyz.reshape(_B * _N, 3), ((0, 0), (0, 125)))
    ws = _fold_weights(params)

    idxg = _cylinder_query(xyzr, ncz, vr9)

    gfeat_i, gxyz = _sc_gather(idxg.reshape(_B * _S * _NS), feat_flat, xyz_pad)
    gfeat = jax.lax.bitcast_convert_type(gfeat_i, jnp.bfloat16).reshape(
        _B * _S * _NS, _C)

    ncf = jnp.repeat(new_xyz.reshape(_B * _S, 3), _NS, axis=0)
    vrf = jnp.repeat(view_rot.reshape(_B * _S, 9), _NS, axis=0)
    preds_f, v_f, bg_f = _mlp(gfeat, gxyz, ncf, vrf, ws)

    preds = preds_f.reshape(_B, _S, _NO)
    v = v_f.reshape(_B, _S, _C).swapaxes(1, 2)
    bg = bg_f.reshape(_B, _S, _C).swapaxes(1, 2)
    return preds, v, bg


# speedup vs baseline: 2.4369x; 2.4369x over previous
import jax
import jax.numpy as jnp
from jax.experimental import pallas as pl
from jax.experimental.pallas import tpu as pltpu
from jax.experimental.pallas import tpu_sc as plsc

_B, _N, _S, _C, _NS = 2, 10000, 512, 512, 16
_RADIUS, _HMIN, _HMAX = 0.05, -0.02, 0.04
_NO = 480
_LANES = 128
_NCH = (_N + _LANES - 1) // _LANES
_NP = _NCH * _LANES
_SB = 128
_MB = 64
_BIG = 2 ** 30
_GW = 64


def _bf(a):
    return a.astype(jnp.bfloat16).astype(jnp.float32)


def _query_body(xyz_ref, nc_ref, vr_ref, out_ref, keys_ref):
    b = pl.program_id(0)
    cx = nc_ref[0, :, 0:1]
    cy = nc_ref[0, :, 1:2]
    cz = nc_ref[0, :, 2:3]
    vr = [_bf(vr_ref[0, :, k:k + 1]) for k in range(9)]
    r2 = jnp.float32(_RADIUS * _RADIUS)
    for c in range(_NCH):
        px = xyz_ref[0, 0, c, :][None, :]
        py = xyz_ref[0, 1, c, :][None, :]
        pz = xyz_ref[0, 2, c, :][None, :]
        relx = _bf(px - cx)
        rely = _bf(py - cy)
        relz = _bf(pz - cz)
        rx = vr[0] * relx + vr[1] * rely + vr[2] * relz
        ry = vr[3] * relx + vr[4] * rely + vr[5] * relz
        rz = vr[6] * relx + vr[7] * rely + vr[8] * relz
        d2 = ry * ry + rz * rz
        iota = jax.lax.broadcasted_iota(jnp.int32, (_SB, _LANES), 1) + c * _LANES
        mask = (d2 <= r2) & (rx >= _HMIN) & (rx <= _HMAX) & (iota < _N)
        keys_ref[:, c * _LANES:(c + 1) * _LANES] = jnp.where(mask, iota, _BIG)
    m = jnp.full((_SB, 1), -1, jnp.int32)
    cols = []
    for _ in range(_NS):
        kall = keys_ref[:, :]
        kk = jnp.where(kall > m, kall, _BIG)
        m = jnp.min(kk, axis=1, keepdims=True)
        cols.append(m)
    K = jnp.concatenate(cols, axis=1)
    fb = jnp.where(K[:, 0:1] < _BIG, K[:, 0:1], 0)
    idx = jnp.where(K < _BIG, K, fb) + b * _N
    out_ref[0] = idx


def _cylinder_query(xyzr, ncz, vr9):
    return pl.pallas_call(
        _query_body,
        grid=(_B, _S // _SB),
        in_specs=[
            pl.BlockSpec((1, 3, _NCH, _LANES), lambda b, j: (b, 0, 0, 0)),
            pl.BlockSpec((1, _SB, 3), lambda b, j: (b, j, 0)),
            pl.BlockSpec((1, _SB, 9), lambda b, j: (b, j, 0)),
        ],
        out_specs=pl.BlockSpec((1, _SB, _NS), lambda b, j: (b, j, 0)),
        out_shape=jax.ShapeDtypeStruct((_B, _S, _NS), jnp.int32),
        scratch_shapes=[pltpu.VMEM((_SB, _NP), jnp.int32)],
    )(xyzr, ncz, vr9)


def _sc_gather(idx_flat, feat_flat, xyz_pad):
    nidx = idx_flat.shape[0]
    nw = 32
    bpw = nidx // nw
    nch = bpw // _GW
    mesh = plsc.VectorSubcoreMesh(core_axis_name="c", subcore_axis_name="s")

    @pl.kernel(
        out_type=(
            jax.ShapeDtypeStruct((nidx, _C), jnp.float32),
            jax.ShapeDtypeStruct((nidx, 128), jnp.float32),
        ),
        mesh=mesh,
        scratch_types=[
            pltpu.VMEM((bpw,), jnp.int32),
            pltpu.VMEM((_GW, _C), jnp.float32),
            pltpu.VMEM((_GW, 128), jnp.float32),
            pltpu.SemaphoreType.DMA,
            pltpu.SemaphoreType.DMA,
        ],
    )
    def k(feat_hbm, xyz_hbm, i_hbm, of_hbm, ox_hbm,
          idx_v, feat_v, xyz_v, sem_f, sem_x):
        wid = jax.lax.axis_index("s") * 2 + jax.lax.axis_index("c")
        base = wid * bpw
        pltpu.sync_copy(i_hbm.at[pl.ds(base, bpw)], idx_v)
        for c in range(nch):
            isub = idx_v.at[pl.ds(c * _GW, _GW)]
            cf = pltpu.async_copy(feat_hbm.at[isub], feat_v, sem_f)
            cx = pltpu.async_copy(xyz_hbm.at[isub], xyz_v, sem_x)
            cf.wait()
            cx.wait()
            pltpu.sync_copy(feat_v, of_hbm.at[pl.ds(base + c * _GW, _GW)])
            pltpu.sync_copy(xyz_v, ox_hbm.at[pl.ds(base + c * _GW, _GW)])

    return k(feat_flat, xyz_pad, idx_flat)


def _mlp_body(gf_ref, gx_ref, nc_ref, vr_ref,
              w0x_ref, w0f_ref, b0_ref, w1_ref, b1_ref, w2_ref, b2_ref,
              c1_ref, d1_ref, c2_ref, d2_ref, c3_ref, d3_ref,
              preds_ref, v_ref, bg_ref):
    hp = jax.lax.Precision.DEFAULT
    inv_r = jnp.float32(1.0 / _RADIUS)
    rel = [_bf(gx_ref[:, i:i + 1] - nc_ref[:, i:i + 1]) for i in range(3)]
    gxn = []
    for i in range(3):
        rot = (_bf(vr_ref[:, 3 * i + 0:3 * i + 1]) * rel[0]
               + _bf(vr_ref[:, 3 * i + 1:3 * i + 2]) * rel[1]
               + _bf(vr_ref[:, 3 * i + 2:3 * i + 3]) * rel[2])
        gxn.append(_bf(rot * inv_r))
    h = jnp.dot(gf_ref[...], w0f_ref[...], precision=hp,
                preferred_element_type=jnp.float32)
    h = h + b0_ref[...]
    for i in range(3):
        h = h + gxn[i] * _bf(w0x_ref[i, :][None, :])
    h = jax.nn.relu(h)
    h = jax.nn.relu(jnp.dot(h, w1_ref[...], precision=hp,
                            preferred_element_type=jnp.float32) + b1_ref[...])
    h = jax.nn.relu(jnp.dot(h, w2_ref[...], precision=hp,
                            preferred_element_type=jnp.float32) + b2_ref[...])
    bg = jnp.max(h.reshape(_MB, _NS, _C), axis=1)
    bg_ref[...] = bg
    v1 = jax.nn.relu(jnp.dot(bg, c1_ref[...], precision=hp,
                             preferred_element_type=jnp.float32) + d1_ref[...])
    v2 = jax.nn.relu(jnp.dot(v1, c2_ref[...], precision=hp,
                             preferred_element_type=jnp.float32) + d2_ref[...])
    v_ref[...] = v2
    preds_ref[...] = jnp.dot(v2, c3_ref[...], precision=hp,
                             preferred_element_type=jnp.float32) + d3_ref[...]


def _mlp(gfeat, gxyz, ncf, vrf, ws):
    bs = _B * _S
    nb = bs // _MB
    full = lambda a: pl.BlockSpec(a.shape, lambda i: (0,) * a.ndim)
    (w0x, w0f, b0, w1, b1, w2, b2, c1, d1, c2, d2, c3, d3) = ws
    return pl.pallas_call(
        _mlp_body,
        grid=(nb,),
        in_specs=[
            pl.BlockSpec((_MB * _NS, _C), lambda i: (i, 0)),
            pl.BlockSpec((_MB * _NS, 128), lambda i: (i, 0)),
            pl.BlockSpec((_MB * _NS, 3), lambda i: (i, 0)),
            pl.BlockSpec((_MB * _NS, 9), lambda i: (i, 0)),
            full(w0x), full(w0f), full(b0), full(w1), full(b1),
            full(w2), full(b2), full(c1), full(d1), full(c2), full(d2),
            full(c3), full(d3),
        ],
        out_specs=[
            pl.BlockSpec((_MB, _NO), lambda i: (i, 0)),
            pl.BlockSpec((_MB, _C), lambda i: (i, 0)),
            pl.BlockSpec((_MB, _C), lambda i: (i, 0)),
        ],
        out_shape=[
            jax.ShapeDtypeStruct((bs, _NO), jnp.float32),
            jax.ShapeDtypeStruct((bs, _C), jnp.float32),
            jax.ShapeDtypeStruct((bs, _C), jnp.float32),
        ],
    )(gfeat, gxyz, ncf, vrf, w0x, w0f, b0, w1, b1, w2, b2,
      c1, d1, c2, d2, c3, d3)


def _fold_weights(params):
    def fold(w, b, g, beta):
        return w * g[:, None], (b * g + beta)[None, :]

    w0, b0 = fold(params['mlp_W0'], params['mlp_b0'],
                  params['mlp_g0'], params['mlp_beta0'])
    w1, b1 = fold(params['mlp_W1'], params['mlp_b1'],
                  params['mlp_g1'], params['mlp_beta1'])
    w2, b2 = fold(params['mlp_W2'], params['mlp_b2'],
                  params['mlp_g2'], params['mlp_beta2'])
    c1, d1 = fold(params['c1_W'], params['c1_b'],
                  params['bn1_g'], params['bn1_b'])
    c2, d2 = fold(params['c2_W'], params['c2_b'],
                  params['bn2_g'], params['bn2_b'])
    return (w0[:, :3].T, w0[:, 3:].T, b0, w1.T, b1, w2.T, b2,
            c1.T, d1, c2.T, d2,
            params['c3_W'].T, params['c3_b'][None, :])


def kernel(xyz, new_xyz, view_rot, features, params):
    xyzr = jnp.pad(xyz, ((0, 0), (0, _NP - _N), (0, 0)),
                   constant_values=1e6).swapaxes(1, 2).reshape(_B, 3, _NCH, _LANES)
    ncz = new_xyz
    vr9 = view_rot.reshape(_B, _S, 9)
    feat_flat = features.swapaxes(1, 2).reshape(_B * _N, _C)
    xyz_pad = jnp.pad(xyz.reshape(_B * _N, 3), ((0, 0), (0, 125)))
    ws = _fold_weights(params)

    idxg = _cylinder_query(xyzr, ncz, vr9)

    gfeat, gxyz = _sc_gather(idxg.reshape(_B * _S * _NS), feat_flat, xyz_pad)

    ncf = jnp.repeat(new_xyz.reshape(_B * _S, 3), _NS, axis=0)
    vrf = jnp.repeat(view_rot.reshape(_B * _S, 9), _NS, axis=0)
    preds_f, v_f, bg_f = _mlp(gfeat, gxyz, ncf, vrf, ws)

    preds = preds_f.reshape(_B, _S, _NO)
    v = v_f.reshape(_B, _S, _C).swapaxes(1, 2)
    bg = bg_f.reshape(_B, _S, _C).swapaxes(1, 2)
    return preds, v, bg


# speedup vs baseline: 2.4690x; 1.0132x over previous
import jax
import jax.numpy as jnp
from jax.experimental import pallas as pl
from jax.experimental.pallas import tpu as pltpu
from jax.experimental.pallas import tpu_sc as plsc

_B, _N, _S, _C, _NS = 2, 10000, 512, 512, 16
_RADIUS, _HMIN, _HMAX = 0.05, -0.02, 0.04
_NO = 480
_LANES = 128
_NCH = (_N + _LANES - 1) // _LANES
_NP = _NCH * _LANES
_SB = 128
_MB = 64
_BIG = 2 ** 30
_GW = 64


def _bf(a):
    return a.astype(jnp.bfloat16).astype(jnp.float32)


def _query_body(xyz_ref, nc_ref, vr_ref, out_ref, keys_ref, *, boff):
    cx = nc_ref[0, :, 0:1]
    cy = nc_ref[0, :, 1:2]
    cz = nc_ref[0, :, 2:3]
    vr = [_bf(vr_ref[0, :, k:k + 1]) for k in range(9)]
    r2 = jnp.float32(_RADIUS * _RADIUS)
    for c in range(_NCH):
        px = xyz_ref[0, 0, c, :][None, :]
        py = xyz_ref[0, 1, c, :][None, :]
        pz = xyz_ref[0, 2, c, :][None, :]
        relx = _bf(px - cx)
        rely = _bf(py - cy)
        relz = _bf(pz - cz)
        rx = vr[0] * relx + vr[1] * rely + vr[2] * relz
        ry = vr[3] * relx + vr[4] * rely + vr[5] * relz
        rz = vr[6] * relx + vr[7] * rely + vr[8] * relz
        d2 = ry * ry + rz * rz
        iota = jax.lax.broadcasted_iota(jnp.int32, (_SB, _LANES), 1) + c * _LANES
        mask = (d2 <= r2) & (rx >= _HMIN) & (rx <= _HMAX) & (iota < _N)
        keys_ref[:, c * _LANES:(c + 1) * _LANES] = jnp.where(mask, iota, _BIG)
    m = jnp.full((_SB, 1), -1, jnp.int32)
    cols = []
    for _ in range(_NS):
        kall = keys_ref[:, :]
        kk = jnp.where(kall > m, kall, _BIG)
        m = jnp.min(kk, axis=1, keepdims=True)
        cols.append(m)
    K = jnp.concatenate(cols, axis=1)
    fb = jnp.where(K[:, 0:1] < _BIG, K[:, 0:1], 0)
    idx = jnp.where(K < _BIG, K, fb) + boff
    out_ref[0] = idx


def _cylinder_query(xyzr_b, ncz_b, vr9_b, boff):
    import functools
    return pl.pallas_call(
        functools.partial(_query_body, boff=boff),
        grid=(1, _S // _SB),
        in_specs=[
            pl.BlockSpec((1, 3, _NCH, _LANES), lambda b, j: (b, 0, 0, 0)),
            pl.BlockSpec((1, _SB, 3), lambda b, j: (b, j, 0)),
            pl.BlockSpec((1, _SB, 9), lambda b, j: (b, j, 0)),
        ],
        out_specs=pl.BlockSpec((1, _SB, _NS), lambda b, j: (b, j, 0)),
        out_shape=jax.ShapeDtypeStruct((1, _S, _NS), jnp.int32),
        scratch_shapes=[pltpu.VMEM((_SB, _NP), jnp.int32)],
    )(xyzr_b, ncz_b, vr9_b)


def _sc_gather(idx_flat, feat_flat, xyz_pad):
    nidx = idx_flat.shape[0]
    nw = 32
    bpw = nidx // nw
    nch = bpw // _GW
    mesh = plsc.VectorSubcoreMesh(core_axis_name="c", subcore_axis_name="s")

    @pl.kernel(
        out_type=(
            jax.ShapeDtypeStruct((nidx, _C), jnp.float32),
            jax.ShapeDtypeStruct((nidx, 128), jnp.float32),
        ),
        mesh=mesh,
        scratch_types=[
            pltpu.VMEM((bpw,), jnp.int32),
            pltpu.VMEM((_GW, _C), jnp.float32),
            pltpu.VMEM((_GW, 128), jnp.float32),
            pltpu.SemaphoreType.DMA,
            pltpu.SemaphoreType.DMA,
        ],
    )
    def k(feat_hbm, xyz_hbm, i_hbm, of_hbm, ox_hbm,
          idx_v, feat_v, xyz_v, sem_f, sem_x):
        wid = jax.lax.axis_index("s") * 2 + jax.lax.axis_index("c")
        base = wid * bpw
        pltpu.sync_copy(i_hbm.at[pl.ds(base, bpw)], idx_v)
        for c in range(nch):
            isub = idx_v.at[pl.ds(c * _GW, _GW)]
            cf = pltpu.async_copy(feat_hbm.at[isub], feat_v, sem_f)
            cx = pltpu.async_copy(xyz_hbm.at[isub], xyz_v, sem_x)
            cf.wait()
            cx.wait()
            pltpu.sync_copy(feat_v, of_hbm.at[pl.ds(base + c * _GW, _GW)])
            pltpu.sync_copy(xyz_v, ox_hbm.at[pl.ds(base + c * _GW, _GW)])

    return k(feat_flat, xyz_pad, idx_flat)


def _mlp_body(gf_ref, gx_ref, nc_ref, vr_ref,
              w0x_ref, w0f_ref, b0_ref, w1_ref, b1_ref, w2_ref, b2_ref,
              c1_ref, d1_ref, c2_ref, d2_ref, c3_ref, d3_ref,
              preds_ref, v_ref, bg_ref):
    hp = jax.lax.Precision.DEFAULT
    inv_r = jnp.float32(1.0 / _RADIUS)
    rel = [_bf(gx_ref[:, i:i + 1] - nc_ref[:, i:i + 1]) for i in range(3)]
    gxn = []
    for i in range(3):
        rot = (_bf(vr_ref[:, 3 * i + 0:3 * i + 1]) * rel[0]
               + _bf(vr_ref[:, 3 * i + 1:3 * i + 2]) * rel[1]
               + _bf(vr_ref[:, 3 * i + 2:3 * i + 3]) * rel[2])
        gxn.append(_bf(rot * inv_r))
    h = jnp.dot(gf_ref[...], w0f_ref[...], precision=hp,
                preferred_element_type=jnp.float32)
    h = h + b0_ref[...]
    for i in range(3):
        h = h + gxn[i] * _bf(w0x_ref[i, :][None, :])
    h = jax.nn.relu(h)
    h = jax.nn.relu(jnp.dot(h, w1_ref[...], precision=hp,
                            preferred_element_type=jnp.float32) + b1_ref[...])
    h = jax.nn.relu(jnp.dot(h, w2_ref[...], precision=hp,
                            preferred_element_type=jnp.float32) + b2_ref[...])
    bg = jnp.max(h.reshape(_MB, _NS, _C), axis=1)
    bg_ref[...] = bg
    v1 = jax.nn.relu(jnp.dot(bg, c1_ref[...], precision=hp,
                             preferred_element_type=jnp.float32) + d1_ref[...])
    v2 = jax.nn.relu(jnp.dot(v1, c2_ref[...], precision=hp,
                             preferred_element_type=jnp.float32) + d2_ref[...])
    v_ref[...] = v2
    preds_ref[...] = jnp.dot(v2, c3_ref[...], precision=hp,
                             preferred_element_type=jnp.float32) + d3_ref[...]


def _mlp(gfeat, gxyz, ncf, vrf, ws):
    bs = gfeat.shape[0] // _NS
    nb = bs // _MB
    full = lambda a: pl.BlockSpec(a.shape, lambda i: (0,) * a.ndim)
    (w0x, w0f, b0, w1, b1, w2, b2, c1, d1, c2, d2, c3, d3) = ws
    return pl.pallas_call(
        _mlp_body,
        grid=(nb,),
        in_specs=[
            pl.BlockSpec((_MB * _NS, _C), lambda i: (i, 0)),
            pl.BlockSpec((_MB * _NS, 128), lambda i: (i, 0)),
            pl.BlockSpec((_MB * _NS, 3), lambda i: (i, 0)),
            pl.BlockSpec((_MB * _NS, 9), lambda i: (i, 0)),
            full(w0x), full(w0f), full(b0), full(w1), full(b1),
            full(w2), full(b2), full(c1), full(d1), full(c2), full(d2),
            full(c3), full(d3),
        ],
        out_specs=[
            pl.BlockSpec((_MB, _NO), lambda i: (i, 0)),
            pl.BlockSpec((_MB, _C), lambda i: (i, 0)),
            pl.BlockSpec((_MB, _C), lambda i: (i, 0)),
        ],
        out_shape=[
            jax.ShapeDtypeStruct((bs, _NO), jnp.float32),
            jax.ShapeDtypeStruct((bs, _C), jnp.float32),
            jax.ShapeDtypeStruct((bs, _C), jnp.float32),
        ],
    )(gfeat, gxyz, ncf, vrf, w0x, w0f, b0, w1, b1, w2, b2,
      c1, d1, c2, d2, c3, d3)


def _fold_weights(params):
    def fold(w, b, g, beta):
        return w * g[:, None], (b * g + beta)[None, :]

    w0, b0 = fold(params['mlp_W0'], params['mlp_b0'],
                  params['mlp_g0'], params['mlp_beta0'])
    w1, b1 = fold(params['mlp_W1'], params['mlp_b1'],
                  params['mlp_g1'], params['mlp_beta1'])
    w2, b2 = fold(params['mlp_W2'], params['mlp_b2'],
                  params['mlp_g2'], params['mlp_beta2'])
    c1, d1 = fold(params['c1_W'], params['c1_b'],
                  params['bn1_g'], params['bn1_b'])
    c2, d2 = fold(params['c2_W'], params['c2_b'],
                  params['bn2_g'], params['bn2_b'])
    return (w0[:, :3].T, w0[:, 3:].T, b0, w1.T, b1, w2.T, b2,
            c1.T, d1, c2.T, d2,
            params['c3_W'].T, params['c3_b'][None, :])


def kernel(xyz, new_xyz, view_rot, features, params):
    xyzr = jnp.pad(xyz, ((0, 0), (0, _NP - _N), (0, 0)),
                   constant_values=1e6).swapaxes(1, 2).reshape(_B, 3, _NCH, _LANES)
    ncz = new_xyz
    vr9 = view_rot.reshape(_B, _S, 9)
    feat_flat = features.swapaxes(1, 2).reshape(_B * _N, _C)
    xyz_pad = jnp.pad(xyz.reshape(_B * _N, 3), ((0, 0), (0, 125)))
    ws = _fold_weights(params)

    ncf = jnp.repeat(new_xyz.reshape(_B * _S, 3), _NS, axis=0)
    vrf = jnp.repeat(view_rot.reshape(_B * _S, 9), _NS, axis=0)
    preds_l, v_l, bg_l = [], [], []
    for b in range(_B):
        idx_b = _cylinder_query(xyzr[b:b + 1], ncz[b:b + 1], vr9[b:b + 1],
                                b * _N)
        gfeat, gxyz = _sc_gather(idx_b.reshape(_S * _NS), feat_flat, xyz_pad)
        pf, vf, bf = _mlp(gfeat, gxyz,
                          ncf[b * _S * _NS:(b + 1) * _S * _NS],
                          vrf[b * _S * _NS:(b + 1) * _S * _NS], ws)
        preds_l.append(pf)
        v_l.append(vf)
        bg_l.append(bf)

    preds = jnp.stack(preds_l).reshape(_B, _S, _NO)
    v = jnp.stack(v_l).reshape(_B, _S, _C).swapaxes(1, 2)
    bg = jnp.stack(bg_l).reshape(_B, _S, _C).swapaxes(1, 2)
    return preds, v, bg


# speedup vs baseline: 2.4864x; 1.0070x over previous
import jax
import jax.numpy as jnp
from jax.experimental import pallas as pl
from jax.experimental.pallas import tpu as pltpu
from jax.experimental.pallas import tpu_sc as plsc

_B, _N, _S, _C, _NS = 2, 10000, 512, 512, 16
_RADIUS, _HMIN, _HMAX = 0.05, -0.02, 0.04
_NO = 480
_LANES = 128
_NCH = (_N + _LANES - 1) // _LANES
_NP = _NCH * _LANES
_SB = 128
_MB = 128
_BIG = 2 ** 30
_GW = 64


def _bf(a):
    return a.astype(jnp.bfloat16).astype(jnp.float32)


def _query_body(xyz_ref, nc_ref, vr_ref, out_ref, keys_ref, *, boff):
    cx = nc_ref[0, :, 0:1]
    cy = nc_ref[0, :, 1:2]
    cz = nc_ref[0, :, 2:3]
    vr = [_bf(vr_ref[0, :, k:k + 1]) for k in range(9)]
    r2 = jnp.float32(_RADIUS * _RADIUS)
    for c in range(_NCH):
        px = xyz_ref[0, 0, c, :][None, :]
        py = xyz_ref[0, 1, c, :][None, :]
        pz = xyz_ref[0, 2, c, :][None, :]
        relx = _bf(px - cx)
        rely = _bf(py - cy)
        relz = _bf(pz - cz)
        rx = vr[0] * relx + vr[1] * rely + vr[2] * relz
        ry = vr[3] * relx + vr[4] * rely + vr[5] * relz
        rz = vr[6] * relx + vr[7] * rely + vr[8] * relz
        d2 = ry * ry + rz * rz
        iota = jax.lax.broadcasted_iota(jnp.int32, (_SB, _LANES), 1) + c * _LANES
        mask = (d2 <= r2) & (rx >= _HMIN) & (rx <= _HMAX) & (iota < _N)
        keys_ref[:, c * _LANES:(c + 1) * _LANES] = jnp.where(mask, iota, _BIG)
    m = jnp.full((_SB, 1), -1, jnp.int32)
    cols = []
    for _ in range(_NS):
        kall = keys_ref[:, :]
        kk = jnp.where(kall > m, kall, _BIG)
        m = jnp.min(kk, axis=1, keepdims=True)
        cols.append(m)
    K = jnp.concatenate(cols, axis=1)
    fb = jnp.where(K[:, 0:1] < _BIG, K[:, 0:1], 0)
    idx = jnp.where(K < _BIG, K, fb) + boff
    out_ref[0] = idx


def _cylinder_query(xyzr_b, ncz_b, vr9_b, boff):
    import functools
    return pl.pallas_call(
        functools.partial(_query_body, boff=boff),
        grid=(1, _S // _SB),
        in_specs=[
            pl.BlockSpec((1, 3, _NCH, _LANES), lambda b, j: (b, 0, 0, 0)),
            pl.BlockSpec((1, _SB, 3), lambda b, j: (b, j, 0)),
            pl.BlockSpec((1, _SB, 9), lambda b, j: (b, j, 0)),
        ],
        out_specs=pl.BlockSpec((1, _SB, _NS), lambda b, j: (b, j, 0)),
        out_shape=jax.ShapeDtypeStruct((1, _S, _NS), jnp.int32),
        scratch_shapes=[pltpu.VMEM((_SB, _NP), jnp.int32)],
    )(xyzr_b, ncz_b, vr9_b)


def _sc_gather(idx_flat, feat_flat, xyz_pad):
    nidx = idx_flat.shape[0]
    nw = 32
    bpw = nidx // nw
    nch = bpw // _GW
    mesh = plsc.VectorSubcoreMesh(core_axis_name="c", subcore_axis_name="s")

    @pl.kernel(
        out_type=(
            jax.ShapeDtypeStruct((nidx, _C), jnp.float32),
            jax.ShapeDtypeStruct((nidx, 128), jnp.float32),
        ),
        mesh=mesh,
        scratch_types=[
            pltpu.VMEM((bpw,), jnp.int32),
            pltpu.VMEM((_GW, _C), jnp.float32),
            pltpu.VMEM((_GW, 128), jnp.float32),
            pltpu.SemaphoreType.DMA,
            pltpu.SemaphoreType.DMA,
        ],
    )
    def k(feat_hbm, xyz_hbm, i_hbm, of_hbm, ox_hbm,
          idx_v, feat_v, xyz_v, sem_f, sem_x):
        wid = jax.lax.axis_index("s") * 2 + jax.lax.axis_index("c")
        base = wid * bpw
        pltpu.sync_copy(i_hbm.at[pl.ds(base, bpw)], idx_v)
        for c in range(nch):
            isub = idx_v.at[pl.ds(c * _GW, _GW)]
            cf = pltpu.async_copy(feat_hbm.at[isub], feat_v, sem_f)
            cx = pltpu.async_copy(xyz_hbm.at[isub], xyz_v, sem_x)
            cf.wait()
            cx.wait()
            pltpu.sync_copy(feat_v, of_hbm.at[pl.ds(base + c * _GW, _GW)])
            pltpu.sync_copy(xyz_v, ox_hbm.at[pl.ds(base + c * _GW, _GW)])

    return k(feat_flat, xyz_pad, idx_flat)


def _mlp_body(gf_ref, gx_ref, nc_ref, vr_ref,
              w0x_ref, w0f_ref, b0_ref, w1_ref, b1_ref, w2_ref, b2_ref,
              c1_ref, d1_ref, c2_ref, d2_ref, c3_ref, d3_ref,
              preds_ref, v_ref, bg_ref):
    hp = jax.lax.Precision.DEFAULT
    inv_r = jnp.float32(1.0 / _RADIUS)
    rel = [_bf(gx_ref[:, i:i + 1] - nc_ref[:, i:i + 1]) for i in range(3)]
    gxn = []
    for i in range(3):
        rot = (_bf(vr_ref[:, 3 * i + 0:3 * i + 1]) * rel[0]
               + _bf(vr_ref[:, 3 * i + 1:3 * i + 2]) * rel[1]
               + _bf(vr_ref[:, 3 * i + 2:3 * i + 3]) * rel[2])
        gxn.append(_bf(rot * inv_r))
    h = jnp.dot(gf_ref[...], w0f_ref[...], precision=hp,
                preferred_element_type=jnp.float32)
    h = h + b0_ref[...]
    for i in range(3):
        h = h + gxn[i] * _bf(w0x_ref[i, :][None, :])
    h = jax.nn.relu(h)
    h = jax.nn.relu(jnp.dot(h, w1_ref[...], precision=hp,
                            preferred_element_type=jnp.float32) + b1_ref[...])
    h = jax.nn.relu(jnp.dot(h, w2_ref[...], precision=hp,
                            preferred_element_type=jnp.float32) + b2_ref[...])
    bg = jnp.max(h.reshape(_MB, _NS, _C), axis=1)
    bg_ref[...] = bg
    v1 = jax.nn.relu(jnp.dot(bg, c1_ref[...], precision=hp,
                             preferred_element_type=jnp.float32) + d1_ref[...])
    v2 = jax.nn.relu(jnp.dot(v1, c2_ref[...], precision=hp,
                             preferred_element_type=jnp.float32) + d2_ref[...])
    v_ref[...] = v2
    preds_ref[...] = jnp.dot(v2, c3_ref[...], precision=hp,
                             preferred_element_type=jnp.float32) + d3_ref[...]


def _mlp(gfeat, gxyz, ncf, vrf, ws):
    bs = gfeat.shape[0] // _NS
    nb = bs // _MB
    full = lambda a: pl.BlockSpec(a.shape, lambda i: (0,) * a.ndim)
    (w0x, w0f, b0, w1, b1, w2, b2, c1, d1, c2, d2, c3, d3) = ws
    return pl.pallas_call(
        _mlp_body,
        grid=(nb,),
        in_specs=[
            pl.BlockSpec((_MB * _NS, _C), lambda i: (i, 0)),
            pl.BlockSpec((_MB * _NS, 128), lambda i: (i, 0)),
            pl.BlockSpec((_MB * _NS, 3), lambda i: (i, 0)),
            pl.BlockSpec((_MB * _NS, 9), lambda i: (i, 0)),
            full(w0x), full(w0f), full(b0), full(w1), full(b1),
            full(w2), full(b2), full(c1), full(d1), full(c2), full(d2),
            full(c3), full(d3),
        ],
        out_specs=[
            pl.BlockSpec((_MB, _NO), lambda i: (i, 0)),
            pl.BlockSpec((_MB, _C), lambda i: (i, 0)),
            pl.BlockSpec((_MB, _C), lambda i: (i, 0)),
        ],
        out_shape=[
            jax.ShapeDtypeStruct((bs, _NO), jnp.float32),
            jax.ShapeDtypeStruct((bs, _C), jnp.float32),
            jax.ShapeDtypeStruct((bs, _C), jnp.float32),
        ],
    )(gfeat, gxyz, ncf, vrf, w0x, w0f, b0, w1, b1, w2, b2,
      c1, d1, c2, d2, c3, d3)


def _fold_weights(params):
    def fold(w, b, g, beta):
        return w * g[:, None], (b * g + beta)[None, :]

    w0, b0 = fold(params['mlp_W0'], params['mlp_b0'],
                  params['mlp_g0'], params['mlp_beta0'])
    w1, b1 = fold(params['mlp_W1'], params['mlp_b1'],
                  params['mlp_g1'], params['mlp_beta1'])
    w2, b2 = fold(params['mlp_W2'], params['mlp_b2'],
                  params['mlp_g2'], params['mlp_beta2'])
    c1, d1 = fold(params['c1_W'], params['c1_b'],
                  params['bn1_g'], params['bn1_b'])
    c2, d2 = fold(params['c2_W'], params['c2_b'],
                  params['bn2_g'], params['bn2_b'])
    return (w0[:, :3].T, w0[:, 3:].T, b0, w1.T, b1, w2.T, b2,
            c1.T, d1, c2.T, d2,
            params['c3_W'].T, params['c3_b'][None, :])


def kernel(xyz, new_xyz, view_rot, features, params):
    xyzr = jnp.pad(xyz, ((0, 0), (0, _NP - _N), (0, 0)),
                   constant_values=1e6).swapaxes(1, 2).reshape(_B, 3, _NCH, _LANES)
    ncz = new_xyz
    vr9 = view_rot.reshape(_B, _S, 9)
    feat_flat = features.swapaxes(1, 2).reshape(_B * _N, _C)
    xyz_pad = jnp.pad(xyz.reshape(_B * _N, 3), ((0, 0), (0, 125)))
    ws = _fold_weights(params)

    ncf = jnp.repeat(new_xyz.reshape(_B * _S, 3), _NS, axis=0)
    vrf = jnp.repeat(view_rot.reshape(_B * _S, 9), _NS, axis=0)
    preds_l, v_l, bg_l = [], [], []
    for b in range(_B):
        idx_b = _cylinder_query(xyzr[b:b + 1], ncz[b:b + 1], vr9[b:b + 1],
                                b * _N)
        gfeat, gxyz = _sc_gather(idx_b.reshape(_S * _NS), feat_flat, xyz_pad)
        pf, vf, bf = _mlp(gfeat, gxyz,
                          ncf[b * _S * _NS:(b + 1) * _S * _NS],
                          vrf[b * _S * _NS:(b + 1) * _S * _NS], ws)
        preds_l.append(pf)
        v_l.append(vf)
        bg_l.append(bf)

    preds = jnp.stack(preds_l).reshape(_B, _S, _NO)
    v = jnp.stack(v_l).reshape(_B, _S, _C).swapaxes(1, 2)
    bg = jnp.stack(bg_l).reshape(_B, _S, _C).swapaxes(1, 2)
    return preds, v, bg


# speedup vs baseline: 2.4886x; 1.0009x over previous
import functools

import jax
import jax.numpy as jnp
from jax.experimental import pallas as pl
from jax.experimental.pallas import tpu as pltpu
from jax.experimental.pallas import tpu_sc as plsc

_B, _N, _S, _C, _NS = 2, 10000, 512, 512, 16
_RADIUS, _HMIN, _HMAX = 0.05, -0.02, 0.04
_NO = 480
_LANES = 128
_NCH = (_N + _LANES - 1) // _LANES
_NP = _NCH * _LANES
_SB = 128
_MB = 128
_BIG = 2 ** 30
_GW = 64


def _bf(a):
    return a.astype(jnp.bfloat16).astype(jnp.float32)


def _query_body(xyz_ref, nc_ref, vr_ref, out_ref, keys_ref, *, boff):
    cx = nc_ref[0, :, 0:1]
    cy = nc_ref[0, :, 1:2]
    cz = nc_ref[0, :, 2:3]
    vr = [_bf(vr_ref[0, :, k:k + 1]) for k in range(9)]
    r2 = jnp.float32(_RADIUS * _RADIUS)
    for c in range(_NCH):
        px = xyz_ref[0, 0, c, :][None, :]
        py = xyz_ref[0, 1, c, :][None, :]
        pz = xyz_ref[0, 2, c, :][None, :]
        relx = _bf(px - cx)
        rely = _bf(py - cy)
        relz = _bf(pz - cz)
        rx = vr[0] * relx + vr[1] * rely + vr[2] * relz
        ry = vr[3] * relx + vr[4] * rely + vr[5] * relz
        rz = vr[6] * relx + vr[7] * rely + vr[8] * relz
        d2 = ry * ry + rz * rz
        iota = jax.lax.broadcasted_iota(jnp.int32, (_SB, _LANES), 1) + c * _LANES
        mask = (d2 <= r2) & (rx >= _HMIN) & (rx <= _HMAX) & (iota < _N)
        keys_ref[:, c * _LANES:(c + 1) * _LANES] = jnp.where(mask, iota, _BIG)
    m = jnp.full((_SB, 1), -1, jnp.int32)
    cols = []
    for _ in range(_NS):
        kall = keys_ref[:, :]
        kk = jnp.where(kall > m, kall, _BIG)
        m = jnp.min(kk, axis=1, keepdims=True)
        cols.append(m)
    K = jnp.concatenate(cols, axis=1)
    fb = jnp.where(K[:, 0:1] < _BIG, K[:, 0:1], 0)
    idx = jnp.where(K < _BIG, K, fb) + boff
    out_ref[0] = idx


def _cylinder_query(xyzr_b, ncz_b, vr9_b, boff):
    return pl.pallas_call(
        functools.partial(_query_body, boff=boff),
        grid=(1, _S // _SB),
        in_specs=[
            pl.BlockSpec((1, 3, _NCH, _LANES), lambda b, j: (b, 0, 0, 0)),
            pl.BlockSpec((1, _SB, 3), lambda b, j: (b, j, 0)),
            pl.BlockSpec((1, _SB, 9), lambda b, j: (b, j, 0)),
        ],
        out_specs=pl.BlockSpec((1, _SB, _NS), lambda b, j: (b, j, 0)),
        out_shape=jax.ShapeDtypeStruct((1, _S, _NS), jnp.int32),
        scratch_shapes=[pltpu.VMEM((_SB, _NP), jnp.int32)],
    )(xyzr_b, ncz_b, vr9_b)


def _sc_gather(idx_flat, feat_flat, xyz_pad):
    nidx = idx_flat.shape[0]
    nw = 32
    bpw = nidx // nw
    nch = bpw // _GW
    mesh = plsc.VectorSubcoreMesh(core_axis_name="c", subcore_axis_name="s")

    @pl.kernel(
        out_type=(
            jax.ShapeDtypeStruct((nidx, _C), jnp.float32),
            jax.ShapeDtypeStruct((nidx, 128), jnp.float32),
        ),
        mesh=mesh,
        scratch_types=[
            pltpu.VMEM((bpw,), jnp.int32),
            pltpu.VMEM((_GW, _C), jnp.float32),
            pltpu.VMEM((_GW, 128), jnp.float32),
            pltpu.SemaphoreType.DMA,
            pltpu.SemaphoreType.DMA,
        ],
    )
    def k(feat_hbm, xyz_hbm, i_hbm, of_hbm, ox_hbm,
          idx_v, feat_v, xyz_v, sem_f, sem_x):
        wid = jax.lax.axis_index("s") * 2 + jax.lax.axis_index("c")
        base = wid * bpw
        pltpu.sync_copy(i_hbm.at[pl.ds(base, bpw)], idx_v)
        for c in range(nch):
            isub = idx_v.at[pl.ds(c * _GW, _GW)]
            cf = pltpu.async_copy(feat_hbm.at[isub], feat_v, sem_f)
            cx = pltpu.async_copy(xyz_hbm.at[isub], xyz_v, sem_x)
            cf.wait()
            cx.wait()
            pltpu.sync_copy(feat_v, of_hbm.at[pl.ds(base + c * _GW, _GW)])
            pltpu.sync_copy(xyz_v, ox_hbm.at[pl.ds(base + c * _GW, _GW)])

    return k(feat_flat, xyz_pad, idx_flat)


def _mlp_body(gf_ref, gx_ref, nc_ref, vr_ref,
              w0x_ref, w0f_ref, b0_ref, w1_ref, b1_ref, w2_ref, b2_ref,
              c1_ref, d1_ref, c2_ref, d2_ref, c3_ref, d3_ref,
              preds_ref, v_ref, bg_ref):
    hp = jax.lax.Precision.DEFAULT
    inv_r = jnp.float32(1.0 / _RADIUS)
    rel = [_bf(gx_ref[:, i:i + 1] - nc_ref[:, i:i + 1]) for i in range(3)]
    gxn = []
    for i in range(3):
        rot = (_bf(vr_ref[:, 3 * i + 0:3 * i + 1]) * rel[0]
               + _bf(vr_ref[:, 3 * i + 1:3 * i + 2]) * rel[1]
               + _bf(vr_ref[:, 3 * i + 2:3 * i + 3]) * rel[2])
        gxn.append(_bf(rot * inv_r))
    h = jnp.dot(gf_ref[...], w0f_ref[...], precision=hp,
                preferred_element_type=jnp.float32)
    h = h + b0_ref[...]
    for i in range(3):
        h = h + gxn[i] * _bf(w0x_ref[i, :][None, :])
    h = jax.nn.relu(h)
    h = jax.nn.relu(jnp.dot(h, w1_ref[...], precision=hp,
                            preferred_element_type=jnp.float32) + b1_ref[...])
    h = jax.nn.relu(jnp.dot(h, w2_ref[...], precision=hp,
                            preferred_element_type=jnp.float32) + b2_ref[...])
    bg = jnp.max(h.reshape(_MB, _NS, _C), axis=1)
    bg_ref[...] = bg
    v1 = jax.nn.relu(jnp.dot(bg, c1_ref[...], precision=hp,
                             preferred_element_type=jnp.float32) + d1_ref[...])
    v2 = jax.nn.relu(jnp.dot(v1, c2_ref[...], precision=hp,
                             preferred_element_type=jnp.float32) + d2_ref[...])
    v_ref[...] = v2
    preds_ref[...] = jnp.dot(v2, c3_ref[...], precision=hp,
                             preferred_element_type=jnp.float32) + d3_ref[...]


def _mlp(gfeat, gxyz, ncf, vrf, ws):
    bs = gfeat.shape[0] // _NS
    nb = bs // _MB
    full = lambda a: pl.BlockSpec(a.shape, lambda i: (0,) * a.ndim)
    (w0x, w0f, b0, w1, b1, w2, b2, c1, d1, c2, d2, c3, d3) = ws
    return pl.pallas_call(
        _mlp_body,
        grid=(nb,),
        in_specs=[
            pl.BlockSpec((_MB * _NS, _C), lambda i: (i, 0)),
            pl.BlockSpec((_MB * _NS, 128), lambda i: (i, 0)),
            pl.BlockSpec((_MB * _NS, 3), lambda i: (i, 0)),
            pl.BlockSpec((_MB * _NS, 9), lambda i: (i, 0)),
            full(w0x), full(w0f), full(b0), full(w1), full(b1),
            full(w2), full(b2), full(c1), full(d1), full(c2), full(d2),
            full(c3), full(d3),
        ],
        out_specs=[
            pl.BlockSpec((_MB, _NO), lambda i: (i, 0)),
            pl.BlockSpec((_MB, _C), lambda i: (i, 0)),
            pl.BlockSpec((_MB, _C), lambda i: (i, 0)),
        ],
        out_shape=[
            jax.ShapeDtypeStruct((bs, _NO), jnp.float32),
            jax.ShapeDtypeStruct((bs, _C), jnp.float32),
            jax.ShapeDtypeStruct((bs, _C), jnp.float32),
        ],
    )(gfeat, gxyz, ncf, vrf, w0x, w0f, b0, w1, b1, w2, b2,
      c1, d1, c2, d2, c3, d3)


def _fold_weights(params):
    def fold(w, b, g, beta):
        return w * g[:, None], (b * g + beta)[None, :]

    w0, b0 = fold(params['mlp_W0'], params['mlp_b0'],
                  params['mlp_g0'], params['mlp_beta0'])
    w1, b1 = fold(params['mlp_W1'], params['mlp_b1'],
                  params['mlp_g1'], params['mlp_beta1'])
    w2, b2 = fold(params['mlp_W2'], params['mlp_b2'],
                  params['mlp_g2'], params['mlp_beta2'])
    c1, d1 = fold(params['c1_W'], params['c1_b'],
                  params['bn1_g'], params['bn1_b'])
    c2, d2 = fold(params['c2_W'], params['c2_b'],
                  params['bn2_g'], params['bn2_b'])
    return (w0[:, :3].T, w0[:, 3:].T, b0, w1.T, b1, w2.T, b2,
            c1.T, d1, c2.T, d2,
            params['c3_W'].T, params['c3_b'][None, :])


def kernel(xyz, new_xyz, view_rot, features, params):
    xyzr = jnp.pad(xyz, ((0, 0), (0, _NP - _N), (0, 0)),
                   constant_values=1e6).swapaxes(1, 2).reshape(_B, 3, _NCH, _LANES)
    ncz = new_xyz
    vr9 = view_rot.reshape(_B, _S, 9)
    feat_flat = features.swapaxes(1, 2).reshape(_B * _N, _C)
    xyz_pad = jnp.pad(xyz.reshape(_B * _N, 3), ((0, 0), (0, 125)))
    ws = _fold_weights(params)

    ncf = jnp.repeat(new_xyz.reshape(_B * _S, 3), _NS, axis=0)
    vrf = jnp.repeat(view_rot.reshape(_B * _S, 9), _NS, axis=0)
    preds_l, v_l, bg_l = [], [], []
    for b in range(_B):
        idx_b = _cylinder_query(xyzr[b:b + 1], ncz[b:b + 1], vr9[b:b + 1],
                                b * _N)
        gfeat, gxyz = _sc_gather(idx_b.reshape(_S * _NS), feat_flat, xyz_pad)
        pf, vf, bf = _mlp(gfeat, gxyz,
                          ncf[b * _S * _NS:(b + 1) * _S * _NS],
                          vrf[b * _S * _NS:(b + 1) * _S * _NS], ws)
        preds_l.append(pf)
        v_l.append(vf)
        bg_l.append(bf)

    preds = jnp.stack(preds_l).reshape(_B, _S, _NO)
    v = jnp.stack(v_l).reshape(_B, _S, _C).swapaxes(1, 2)
    bg = jnp.stack(bg_l).reshape(_B, _S, _C).swapaxes(1, 2)
    return preds, v, bg
